# baseline (speedup 1.0000x reference)
import dataclasses
import functools

import jax
import jax.numpy as jnp
from jax import lax
from jax.experimental import pallas as pl
from jax.experimental.pallas import tpu as pltpu
from jax.experimental.pallas import tpu_sc as plsc

L = 16
NC = 2
NS = 16
K = 48
RZ = 40


@functools.lru_cache(maxsize=None)
def _build(NP: int, CH: int, n_chunks: int):
    mesh = plsc.VectorSubcoreMesh(core_axis_name="c", subcore_axis_name="s")
    cp = pltpu.CompilerParams()
    if "needs_layout_passes" in pltpu.CompilerParams.__dataclass_fields__:
        cp = dataclasses.replace(cp, needs_layout_passes=False)

    @functools.partial(
        pl.kernel,
        out_type=jax.ShapeDtypeStruct((NC * NP, CH), jnp.float32),
        mesh=mesh,
        compiler_params=cp,
        scratch_types=[
            [pltpu.VMEM((3 * K,), jnp.int32) for _ in range(2)],
            [pltpu.VMEM((K,), jnp.float32) for _ in range(2)],
            [pltpu.VMEM((K,), jnp.int32) for _ in range(2)],
            [pltpu.VMEM((K,), jnp.int32) for _ in range(2)],
            [pltpu.VMEM((K,), jnp.int32) for _ in range(4)],
            [pltpu.VMEM((K,), jnp.int32) for _ in range(4)],
            [pltpu.VMEM((K, 128), jnp.float32) for _ in range(2)],
            [pltpu.VMEM((K, 128), jnp.float32) for _ in range(2)],
            pltpu.VMEM((K, 128), jnp.float32),
            pltpu.VMEM((K, 128), jnp.float32),
            pltpu.VMEM_SHARED((NP, 128), jnp.float32),
            [pltpu.SemaphoreType.DMA for _ in range(2)],
            [pltpu.SemaphoreType.DMA for _ in range(2)],
            [pltpu.SemaphoreType.DMA for _ in range(2)],
        ],
    )
    def graph_lap(xt_hbm, pk_hbm, out_hbm,
                  pk_v, w2_v, gi_v, gj_v, si_v, sj_v,
                  gA, gB, dA, dB, acc_sh, sem_g, sem_ix, sem_sc):
        cid = lax.axis_index("c")
        sid = lax.axis_index("s")
        zeros = jnp.zeros((L,), jnp.float32)
        off = cid * NP

        def load_idx_sync(k, s):
            g = sid * n_chunks + k
            pltpu.sync_copy(pk_hbm.at[pl.ds(g * 3 * K, 3 * K)], pk_v[s])

        def load_idx_async(k, s):
            g = sid * n_chunks + k
            pltpu.async_copy(pk_hbm.at[pl.ds(g * 3 * K, 3 * K)], pk_v[s], sem_ix[s])

        def wait_idx(s):
            pltpu.make_async_copy(pk_hbm.at[pl.ds(0, 3 * K)], pk_v[s], sem_ix[s]).wait()

        def prep(s, s4):
            @pl.loop(0, K // L)
            def _(t):
                sl = pl.ds(t * L, L)
                iv = pk_v[s][sl]
                jv = pk_v[s][pl.ds(K + t * L, L)]
                wv = plsc.bitcast(pk_v[s][pl.ds(2 * K + t * L, L)], jnp.float32)
                w2_v[s][sl] = wv * wv
                gi_v[s][sl] = iv + off
                gj_v[s][sl] = jv + off
                si_v[s4][sl] = iv
                sj_v[s4][sl] = jv

        H = K // 2

        def fire_gather(s):
            pltpu.async_copy(xt_hbm.at[gi_v[s].at[pl.ds(0, H)]],
                             gA[s].at[pl.ds(0, H)], sem_g[s])
            pltpu.async_copy(xt_hbm.at[gj_v[s].at[pl.ds(0, H)]],
                             gB[s].at[pl.ds(0, H)], sem_g[s])
            pltpu.async_copy(xt_hbm.at[gi_v[s].at[pl.ds(H, H)]],
                             gA[s].at[pl.ds(H, H)], sem_g[s])
            pltpu.async_copy(xt_hbm.at[gj_v[s].at[pl.ds(H, H)]],
                             gB[s].at[pl.ds(H, H)], sem_g[s])

        def wait_gather(s):
            pltpu.make_async_copy(xt_hbm.at[gi_v[s].at[pl.ds(0, H)]],
                                  gA[s].at[pl.ds(0, H)], sem_g[s]).wait()
            pltpu.make_async_copy(xt_hbm.at[gj_v[s].at[pl.ds(0, H)]],
                                  gB[s].at[pl.ds(0, H)], sem_g[s]).wait()
            pltpu.make_async_copy(xt_hbm.at[gi_v[s].at[pl.ds(H, H)]],
                                  gA[s].at[pl.ds(H, H)], sem_g[s]).wait()
            pltpu.make_async_copy(xt_hbm.at[gj_v[s].at[pl.ds(H, H)]],
                                  gB[s].at[pl.ds(H, H)], sem_g[s]).wait()

        def fire_scatter(s, s4):
            pltpu.async_copy(dA, acc_sh.at[si_v[s4]], sem_sc[s], add=True)
            pltpu.async_copy(dB, acc_sh.at[sj_v[s4]], sem_sc[s], add=True)

        def wait_scatter(s, s4):
            pltpu.make_async_copy(dA, acc_sh.at[si_v[s4]], sem_sc[s]).wait()
            pltpu.make_async_copy(dB, acc_sh.at[sj_v[s4]], sem_sc[s]).wait()

        def compute(s):
            w2r = w2_v[s]
            gAr = gA[s]
            gBr = gB[s]

            @pl.loop(0, K, step=2)
            def _(e0):
                e1 = e0 + 1
                w2b0 = plsc.load_gather(w2r, [jnp.zeros((L,), jnp.int32) + e0])
                w2b1 = plsc.load_gather(w2r, [jnp.zeros((L,), jnp.int32) + e1])

                def emit(e, w2b, c, a, b):
                    sl = pl.ds(c * L, L)
                    d = w2b * (a - b)
                    dA[e, sl] = d
                    dB[e, sl] = -d

                prev = None
                for c in range(128 // L):
                    sl = pl.ds(c * L, L)
                    a0 = gAr[e0, sl]
                    b0 = gBr[e0, sl]
                    a1 = gAr[e1, sl]
                    b1 = gBr[e1, sl]
                    if prev is not None:
                        pc, pa0, pb0, pa1, pb1 = prev
                        emit(e0, w2b0, pc, pa0, pb0)
                        emit(e1, w2b1, pc, pa1, pb1)
                    prev = (c, a0, b0, a1, b1)
                pc, pa0, pb0, pa1, pb1 = prev
                emit(e0, w2b0, pc, pa0, pb0)
                emit(e1, w2b1, pc, pa1, pb1)

        @pl.loop(0, RZ)
        def _(r):
            for c in range(128 // L):
                dA[r, pl.ds(c * L, L)] = zeros

        rows_per_tile = NP // NS
        @pl.loop(0, rows_per_tile // RZ)
        def _(k):
            pltpu.sync_copy(
                dA.at[pl.ds(0, RZ)],
                acc_sh.at[pl.ds(sid * rows_per_tile + k * RZ, RZ)])

        load_idx_sync(0, 0)
        load_idx_sync(1, 1)
        prep(0, 0)
        fire_gather(0)
        plsc.subcore_barrier()

        @pl.loop(0, n_chunks // 4)
        def _(k4):
            for p in range(4):
                k = k4 * 4 + p
                p2 = p % 2
                q2 = 1 - p2

                wait_gather(p2)

                @pl.when(jnp.logical_and(k >= 1, k + 1 < n_chunks))
                def _():
                    wait_idx(q2)

                @pl.when(k + 1 < n_chunks)
                def _():
                    prep(q2, (p + 1) % 4)
                    fire_gather(q2)

                @pl.when(k + 2 < n_chunks)
                def _():
                    load_idx_async(k + 2, p2)

                @pl.when(k >= 1)
                def _():
                    wait_scatter(q2, (p + 3) % 4)

                compute(p2)
                fire_scatter(p2, p)

        wait_scatter((n_chunks - 1) % 2, (n_chunks - 1) % 4)
        plsc.subcore_barrier()

        @pl.loop(0, rows_per_tile // RZ)
        def _(k):
            r0 = sid * rows_per_tile + k * RZ
            pltpu.sync_copy(acc_sh.at[pl.ds(r0, RZ)], dA.at[pl.ds(0, RZ)])
            pltpu.sync_copy(dA.at[pl.ds(0, RZ)],
                            out_hbm.at[pl.ds(cid * NP + r0, RZ)])

    return graph_lap


def kernel(x, W, iInd, jInd):
    B, C, N = x.shape
    E = iInd.shape[0]
    CH = C // NC

    n_chunks = -(-E // (NS * K))
    n_chunks = -(-n_chunks // 4) * 4
    E_pad = NS * K * n_chunks
    pad = E_pad - E
    ii = jnp.concatenate([iInd.astype(jnp.int32), jnp.zeros((pad,), jnp.int32)])
    jj = jnp.concatenate([jInd.astype(jnp.int32), jnp.zeros((pad,), jnp.int32)])
    w = jnp.concatenate([W[0, 0].astype(jnp.float32), jnp.zeros((pad,), jnp.float32)])

    ntc = NS * n_chunks
    wbits = jax.lax.bitcast_convert_type(w, jnp.int32)
    pk = jnp.concatenate(
        [ii.reshape(ntc, K), jj.reshape(ntc, K), wbits.reshape(ntc, K)],
        axis=1).reshape(-1)

    NP = -(-N // (NS * RZ)) * (NS * RZ)
    xt = x[0].T.reshape(N, NC, CH).transpose(1, 0, 2).reshape(NC * N, CH)
    xt = jnp.pad(xt.reshape(NC, N, CH), ((0, 0), (0, NP - N), (0, 0))).reshape(NC * NP, CH)

    out2 = _build(NP, CH, n_chunks)(xt, pk)
    return out2.reshape(NC, NP, CH)[:, :N].transpose(0, 2, 1).reshape(1, C, N)


# speedup vs baseline: 1.0004x; 1.0004x over previous
import dataclasses
import functools

import jax
import jax.numpy as jnp
from jax import lax
from jax.experimental import pallas as pl
from jax.experimental.pallas import tpu as pltpu
from jax.experimental.pallas import tpu_sc as plsc

L = 16
NC = 2
NS = 16
K = 48
RZ = 40


@functools.lru_cache(maxsize=None)
def _build(NP: int, CH: int, n_chunks: int):
    mesh = plsc.VectorSubcoreMesh(core_axis_name="c", subcore_axis_name="s")
    cp = pltpu.CompilerParams()
    if "needs_layout_passes" in pltpu.CompilerParams.__dataclass_fields__:
        cp = dataclasses.replace(cp, needs_layout_passes=False)

    @functools.partial(
        pl.kernel,
        out_type=jax.ShapeDtypeStruct((NC * NP, CH), jnp.float32),
        mesh=mesh,
        compiler_params=cp,
        scratch_types=[
            [pltpu.VMEM((3 * K,), jnp.int32) for _ in range(2)],
            [pltpu.VMEM((K,), jnp.float32) for _ in range(2)],
            [pltpu.VMEM((K,), jnp.int32) for _ in range(2)],
            [pltpu.VMEM((K,), jnp.int32) for _ in range(2)],
            [pltpu.VMEM((K,), jnp.int32) for _ in range(4)],
            [pltpu.VMEM((K,), jnp.int32) for _ in range(4)],
            [pltpu.VMEM((K, 128), jnp.float32) for _ in range(2)],
            [pltpu.VMEM((K, 128), jnp.float32) for _ in range(2)],
            pltpu.VMEM((K, 128), jnp.float32),
            pltpu.VMEM((K, 128), jnp.float32),
            pltpu.VMEM_SHARED((NP, 128), jnp.float32),
            [pltpu.SemaphoreType.DMA for _ in range(2)],
            [pltpu.SemaphoreType.DMA for _ in range(2)],
            [pltpu.SemaphoreType.DMA for _ in range(2)],
        ],
    )
    def graph_lap(xt_hbm, pk_hbm, out_hbm,
                  pk_v, w2_v, gi_v, gj_v, si_v, sj_v,
                  gA, gB, dA, dB, acc_sh, sem_g, sem_ix, sem_sc):
        cid = lax.axis_index("c")
        sid = lax.axis_index("s")
        zeros = jnp.zeros((L,), jnp.float32)
        off = cid * NP

        def load_idx_sync(k, s):
            g = sid * n_chunks + k
            pltpu.sync_copy(pk_hbm.at[pl.ds(g * 3 * K, 3 * K)], pk_v[s])

        def load_idx_async(k, s):
            g = sid * n_chunks + k
            pltpu.async_copy(pk_hbm.at[pl.ds(g * 3 * K, 3 * K)], pk_v[s], sem_ix[s])

        def wait_idx(s):
            pltpu.make_async_copy(pk_hbm.at[pl.ds(0, 3 * K)], pk_v[s], sem_ix[s]).wait()

        def prep(s, s4):
            @pl.loop(0, K // L)
            def _(t):
                sl = pl.ds(t * L, L)
                iv = pk_v[s][sl]
                jv = pk_v[s][pl.ds(K + t * L, L)]
                wv = plsc.bitcast(pk_v[s][pl.ds(2 * K + t * L, L)], jnp.float32)
                w2_v[s][sl] = wv * wv
                gi_v[s][sl] = iv + off
                gj_v[s][sl] = jv + off
                si_v[s4][sl] = iv
                sj_v[s4][sl] = jv

        def fire_gather(s):
            pltpu.async_copy(xt_hbm.at[gi_v[s]], gA[s], sem_g[s])
            pltpu.async_copy(xt_hbm.at[gj_v[s]], gB[s], sem_g[s])

        def wait_gather(s):
            pltpu.make_async_copy(xt_hbm.at[gi_v[s]], gA[s], sem_g[s]).wait()
            pltpu.make_async_copy(xt_hbm.at[gj_v[s]], gB[s], sem_g[s]).wait()

        def fire_scatter(s, s4):
            pltpu.async_copy(dA, acc_sh.at[si_v[s4]], sem_sc[s], add=True)
            pltpu.async_copy(dB, acc_sh.at[sj_v[s4]], sem_sc[s], add=True)

        def wait_scatter(s, s4):
            pltpu.make_async_copy(dA, acc_sh.at[si_v[s4]], sem_sc[s]).wait()
            pltpu.make_async_copy(dB, acc_sh.at[sj_v[s4]], sem_sc[s]).wait()

        def compute(s):
            w2r = w2_v[s]
            gAr = gA[s]
            gBr = gB[s]

            @pl.loop(0, K, step=2)
            def _(e0):
                e1 = e0 + 1
                w2b0 = plsc.load_gather(w2r, [jnp.zeros((L,), jnp.int32) + e0])
                w2b1 = plsc.load_gather(w2r, [jnp.zeros((L,), jnp.int32) + e1])

                def emit(e, w2b, c, a, b):
                    sl = pl.ds(c * L, L)
                    d = w2b * (a - b)
                    dA[e, sl] = d
                    dB[e, sl] = -d

                prev = None
                for c in range(128 // L):
                    sl = pl.ds(c * L, L)
                    a0 = gAr[e0, sl]
                    b0 = gBr[e0, sl]
                    a1 = gAr[e1, sl]
                    b1 = gBr[e1, sl]
                    if prev is not None:
                        pc, pa0, pb0, pa1, pb1 = prev
                        emit(e0, w2b0, pc, pa0, pb0)
                        emit(e1, w2b1, pc, pa1, pb1)
                    prev = (c, a0, b0, a1, b1)
                pc, pa0, pb0, pa1, pb1 = prev
                emit(e0, w2b0, pc, pa0, pb0)
                emit(e1, w2b1, pc, pa1, pb1)

        @pl.loop(0, RZ)
        def _(r):
            for c in range(128 // L):
                dA[r, pl.ds(c * L, L)] = zeros

        rows_per_tile = NP // NS
        @pl.loop(0, rows_per_tile // RZ)
        def _(k):
            pltpu.sync_copy(
                dA.at[pl.ds(0, RZ)],
                acc_sh.at[pl.ds(sid * rows_per_tile + k * RZ, RZ)])

        load_idx_sync(0, 0)
        load_idx_sync(1, 1)
        prep(0, 0)
        fire_gather(0)
        plsc.subcore_barrier()

        @pl.loop(0, n_chunks // 4)
        def _(k4):
            for p in range(4):
                k = k4 * 4 + p
                p2 = p % 2
                q2 = 1 - p2

                wait_gather(p2)

                @pl.when(jnp.logical_and(k >= 1, k + 1 < n_chunks))
                def _():
                    wait_idx(q2)

                @pl.when(k + 1 < n_chunks)
                def _():
                    prep(q2, (p + 1) % 4)
                    fire_gather(q2)

                @pl.when(k + 2 < n_chunks)
                def _():
                    load_idx_async(k + 2, p2)

                @pl.when(k >= 1)
                def _():
                    wait_scatter(q2, (p + 3) % 4)

                compute(p2)
                fire_scatter(p2, p)

        wait_scatter((n_chunks - 1) % 2, (n_chunks - 1) % 4)
        plsc.subcore_barrier()

        @pl.loop(0, rows_per_tile // RZ)
        def _(k):
            r0 = sid * rows_per_tile + k * RZ
            pltpu.sync_copy(acc_sh.at[pl.ds(r0, RZ)], dA.at[pl.ds(0, RZ)])
            pltpu.sync_copy(dA.at[pl.ds(0, RZ)],
                            out_hbm.at[pl.ds(cid * NP + r0, RZ)])

    return graph_lap


def kernel(x, W, iInd, jInd):
    B, C, N = x.shape
    E = iInd.shape[0]
    CH = C // NC

    n_chunks = -(-E // (NS * K))
    n_chunks = -(-n_chunks // 4) * 4
    E_pad = NS * K * n_chunks
    pad = E_pad - E
    ii = jnp.concatenate([iInd.astype(jnp.int32), jnp.zeros((pad,), jnp.int32)])
    jj = jnp.concatenate([jInd.astype(jnp.int32), jnp.zeros((pad,), jnp.int32)])
    w = jnp.concatenate([W[0, 0].astype(jnp.float32), jnp.zeros((pad,), jnp.float32)])

    ntc = NS * n_chunks
    wbits = jax.lax.bitcast_convert_type(w, jnp.int32)
    pk = jnp.concatenate(
        [ii.reshape(ntc, K), jj.reshape(ntc, K), wbits.reshape(ntc, K)],
        axis=1).reshape(-1)

    NP = -(-N // (NS * RZ)) * (NS * RZ)
    xt = x[0].T.reshape(N, NC, CH).transpose(1, 0, 2).reshape(NC * N, CH)
    xt = jnp.pad(xt.reshape(NC, N, CH), ((0, 0), (0, NP - N), (0, 0))).reshape(NC * NP, CH)

    out2 = _build(NP, CH, n_chunks)(xt, pk)
    return out2.reshape(NC, NP, CH)[:, :N].transpose(0, 2, 1).reshape(1, C, N)


# speedup vs baseline: 1.1022x; 1.1018x over previous
import dataclasses
import functools

import jax
import jax.numpy as jnp
from jax import lax
from jax.experimental import pallas as pl
from jax.experimental.pallas import tpu as pltpu
from jax.experimental.pallas import tpu_sc as plsc

L = 16
NC = 2
NS = 16
K = 48
RZ = 40


@functools.lru_cache(maxsize=None)
def _build(NP: int, CH: int, n_chunks: int):
    mesh = plsc.VectorSubcoreMesh(core_axis_name="c", subcore_axis_name="s")
    cp = pltpu.CompilerParams()
    if "needs_layout_passes" in pltpu.CompilerParams.__dataclass_fields__:
        cp = dataclasses.replace(cp, needs_layout_passes=False)

    @functools.partial(
        pl.kernel,
        out_type=jax.ShapeDtypeStruct((NC * NP, CH), jnp.float32),
        mesh=mesh,
        compiler_params=cp,
        scratch_types=[
            [pltpu.VMEM((K,), jnp.int32) for _ in range(2)],
            [pltpu.VMEM((K,), jnp.int32) for _ in range(2)],
            [pltpu.VMEM((K,), jnp.float32) for _ in range(2)],
            [pltpu.VMEM((K,), jnp.float32) for _ in range(2)],
            [pltpu.VMEM((K,), jnp.int32) for _ in range(2)],
            [pltpu.VMEM((K,), jnp.int32) for _ in range(2)],
            [pltpu.VMEM((K,), jnp.int32) for _ in range(4)],
            [pltpu.VMEM((K,), jnp.int32) for _ in range(4)],
            [pltpu.VMEM((K, 128), jnp.float32) for _ in range(2)],
            [pltpu.VMEM((K, 128), jnp.float32) for _ in range(2)],
            pltpu.VMEM((K, 128), jnp.float32),
            pltpu.VMEM((K, 128), jnp.float32),
            pltpu.VMEM_SHARED((NP, 128), jnp.float32),
            [pltpu.SemaphoreType.DMA for _ in range(2)],
            [pltpu.SemaphoreType.DMA for _ in range(2)],
            [pltpu.SemaphoreType.DMA for _ in range(2)],
        ],
    )
    def graph_lap(xt_hbm, w_hbm, ii_hbm, jj_hbm, out_hbm,
                  ii_v, jj_v, w_v, w2_v, gi_v, gj_v, si_v, sj_v,
                  gA, gB, dA, dB, acc_sh, sem_g, sem_ix, sem_sc):
        cid = lax.axis_index("c")
        sid = lax.axis_index("s")
        zeros = jnp.zeros((L,), jnp.float32)
        off = cid * NP

        def load_idx_sync(k, s):
            base = (sid * n_chunks + k) * K
            pltpu.sync_copy(ii_hbm.at[pl.ds(base, K)], ii_v[s])
            pltpu.sync_copy(jj_hbm.at[pl.ds(base, K)], jj_v[s])
            pltpu.sync_copy(w_hbm.at[pl.ds(base, K)], w_v[s])

        def load_idx_async(k, s):
            base = (sid * n_chunks + k) * K
            pltpu.async_copy(ii_hbm.at[pl.ds(base, K)], ii_v[s], sem_ix[s])
            pltpu.async_copy(jj_hbm.at[pl.ds(base, K)], jj_v[s], sem_ix[s])
            pltpu.async_copy(w_hbm.at[pl.ds(base, K)], w_v[s], sem_ix[s])

        def wait_idx(s):
            pltpu.make_async_copy(ii_hbm.at[pl.ds(0, K)], ii_v[s], sem_ix[s]).wait()
            pltpu.make_async_copy(jj_hbm.at[pl.ds(0, K)], jj_v[s], sem_ix[s]).wait()
            pltpu.make_async_copy(w_hbm.at[pl.ds(0, K)], w_v[s], sem_ix[s]).wait()

        def prep(s, s4):
            @pl.loop(0, K // L)
            def _(t):
                sl = pl.ds(t * L, L)
                wv = w_v[s][sl]
                w2_v[s][sl] = wv * wv
                iv = ii_v[s][sl]
                jv = jj_v[s][sl]
                gi_v[s][sl] = iv + off
                gj_v[s][sl] = jv + off
                si_v[s4][sl] = iv
                sj_v[s4][sl] = jv

        def fire_gather(s):
            pltpu.async_copy(xt_hbm.at[gi_v[s]], gA[s], sem_g[s])
            pltpu.async_copy(xt_hbm.at[gj_v[s]], gB[s], sem_g[s])

        def wait_gather(s):
            pltpu.make_async_copy(xt_hbm.at[gi_v[s]], gA[s], sem_g[s]).wait()
            pltpu.make_async_copy(xt_hbm.at[gj_v[s]], gB[s], sem_g[s]).wait()

        def fire_scatter(s, s4):
            pltpu.async_copy(dA, acc_sh.at[si_v[s4]], sem_sc[s], add=True)
            pltpu.async_copy(dB, acc_sh.at[sj_v[s4]], sem_sc[s], add=True)

        def wait_scatter(s, s4):
            pltpu.make_async_copy(dA, acc_sh.at[si_v[s4]], sem_sc[s]).wait()
            pltpu.make_async_copy(dB, acc_sh.at[sj_v[s4]], sem_sc[s]).wait()

        def compute(s):
            w2r = w2_v[s]
            gAr = gA[s]
            gBr = gB[s]
            NCH = 128 // L

            @pl.loop(0, K, step=2)
            def _(e0):
                e1 = e0 + 1
                w2b0 = plsc.load_gather(w2r, [jnp.zeros((L,), jnp.int32) + e0])
                w2b1 = plsc.load_gather(w2r, [jnp.zeros((L,), jnp.int32) + e1])

                def emit(e, w2b, c, a, b):
                    sl = pl.ds(c * L, L)
                    d = w2b * (a - b)
                    dA[e, sl] = d
                    dB[e, sl] = -d

                prev = None
                for c in range(NCH):
                    sl = pl.ds(c * L, L)
                    a0 = gAr[e0, sl]
                    b0 = gBr[e0, sl]
                    a1 = gAr[e1, sl]
                    b1 = gBr[e1, sl]
                    if prev is not None:
                        pc, pa0, pb0, pa1, pb1 = prev
                        emit(e0, w2b0, pc, pa0, pb0)
                        emit(e1, w2b1, pc, pa1, pb1)
                    prev = (c, a0, b0, a1, b1)
                pc, pa0, pb0, pa1, pb1 = prev
                emit(e0, w2b0, pc, pa0, pb0)
                emit(e1, w2b1, pc, pa1, pb1)

        @pl.loop(0, RZ)
        def _(r):
            for c in range(128 // L):
                dA[r, pl.ds(c * L, L)] = zeros

        rows_per_tile = NP // NS
        @pl.loop(0, rows_per_tile // RZ)
        def _(k):
            pltpu.sync_copy(
                dA.at[pl.ds(0, RZ)],
                acc_sh.at[pl.ds(sid * rows_per_tile + k * RZ, RZ)])

        load_idx_sync(0, 0)
        load_idx_sync(1, 1)
        prep(0, 0)
        fire_gather(0)
        plsc.subcore_barrier()

        @pl.loop(0, n_chunks // 4)
        def _(k4):
            for p in range(4):
                k = k4 * 4 + p
                p2 = p % 2
                q2 = 1 - p2

                wait_gather(p2)

                @pl.when(jnp.logical_and(k >= 1, k + 1 < n_chunks))
                def _():
                    wait_idx(q2)

                @pl.when(k + 1 < n_chunks)
                def _():
                    prep(q2, (p + 1) % 4)
                    fire_gather(q2)

                @pl.when(k + 2 < n_chunks)
                def _():
                    load_idx_async(k + 2, p2)

                @pl.when(k >= 1)
                def _():
                    wait_scatter(q2, (p + 3) % 4)

                compute(p2)
                fire_scatter(p2, p)

        wait_scatter((n_chunks - 1) % 2, (n_chunks - 1) % 4)
        plsc.subcore_barrier()

        @pl.loop(0, rows_per_tile // RZ)
        def _(k):
            r0 = sid * rows_per_tile + k * RZ
            pltpu.sync_copy(acc_sh.at[pl.ds(r0, RZ)], dA.at[pl.ds(0, RZ)])
            pltpu.sync_copy(dA.at[pl.ds(0, RZ)],
                            out_hbm.at[pl.ds(cid * NP + r0, RZ)])

    return graph_lap


def kernel(x, W, iInd, jInd):
    B, C, N = x.shape
    E = iInd.shape[0]
    CH = C // NC

    n_chunks = -(-E // (NS * K))
    n_chunks = -(-n_chunks // 4) * 4
    E_pad = NS * K * n_chunks
    pad = E_pad - E
    ii = jnp.concatenate([iInd.astype(jnp.int32), jnp.zeros((pad,), jnp.int32)])
    jj = jnp.concatenate([jInd.astype(jnp.int32), jnp.zeros((pad,), jnp.int32)])
    w = jnp.concatenate([W[0, 0].astype(jnp.float32), jnp.zeros((pad,), jnp.float32)])

    NP = -(-N // (NS * RZ)) * (NS * RZ)
    xt = x[0].T.reshape(N, NC, CH).transpose(1, 0, 2).reshape(NC * N, CH)
    xt = jnp.pad(xt.reshape(NC, N, CH), ((0, 0), (0, NP - N), (0, 0))).reshape(NC * NP, CH)

    out2 = _build(NP, CH, n_chunks)(xt, w, ii, jj)
    return out2.reshape(NC, NP, CH)[:, :N].transpose(0, 2, 1).reshape(1, C, N)


# speedup vs baseline: 1.1028x; 1.0005x over previous
import dataclasses
import functools

import jax
import jax.numpy as jnp
from jax import lax
from jax.experimental import pallas as pl
from jax.experimental.pallas import tpu as pltpu
from jax.experimental.pallas import tpu_sc as plsc

L = 16
NC = 2
NS = 16
K = 48
RZ = 40


@functools.lru_cache(maxsize=None)
def _build(NP: int, CH: int, n_chunks: int):
    mesh = plsc.VectorSubcoreMesh(core_axis_name="c", subcore_axis_name="s")
    cp = pltpu.CompilerParams()
    if "needs_layout_passes" in pltpu.CompilerParams.__dataclass_fields__:
        cp = dataclasses.replace(cp, needs_layout_passes=False)

    @functools.partial(
        pl.kernel,
        out_type=jax.ShapeDtypeStruct((NC * NP, CH), jnp.float32),
        mesh=mesh,
        compiler_params=cp,
        scratch_types=[
            [pltpu.VMEM((K,), jnp.int32) for _ in range(2)],
            [pltpu.VMEM((K,), jnp.int32) for _ in range(2)],
            [pltpu.VMEM((K,), jnp.float32) for _ in range(2)],
            [pltpu.VMEM((K,), jnp.float32) for _ in range(2)],
            [pltpu.VMEM((2 * K,), jnp.int32) for _ in range(2)],
            [pltpu.VMEM((K,), jnp.int32) for _ in range(4)],
            [pltpu.VMEM((K,), jnp.int32) for _ in range(4)],
            [pltpu.VMEM((2 * K, 128), jnp.float32) for _ in range(2)],
            pltpu.VMEM((K, 128), jnp.float32),
            pltpu.VMEM((K, 128), jnp.float32),
            pltpu.VMEM_SHARED((NP, 128), jnp.float32),
            [pltpu.SemaphoreType.DMA for _ in range(2)],
            [pltpu.SemaphoreType.DMA for _ in range(2)],
            [pltpu.SemaphoreType.DMA for _ in range(2)],
        ],
    )
    def graph_lap(xt_hbm, w_hbm, ii_hbm, jj_hbm, out_hbm,
                  ii_v, jj_v, w_v, w2_v, gij_v, si_v, sj_v,
                  gAB, dA, dB, acc_sh, sem_g, sem_ix, sem_sc):
        cid = lax.axis_index("c")
        sid = lax.axis_index("s")
        zeros = jnp.zeros((L,), jnp.float32)
        off = cid * NP

        def load_idx_sync(k, s):
            base = (sid * n_chunks + k) * K
            pltpu.sync_copy(ii_hbm.at[pl.ds(base, K)], ii_v[s])
            pltpu.sync_copy(jj_hbm.at[pl.ds(base, K)], jj_v[s])
            pltpu.sync_copy(w_hbm.at[pl.ds(base, K)], w_v[s])

        def load_idx_async(k, s):
            base = (sid * n_chunks + k) * K
            pltpu.async_copy(ii_hbm.at[pl.ds(base, K)], ii_v[s], sem_ix[s])
            pltpu.async_copy(jj_hbm.at[pl.ds(base, K)], jj_v[s], sem_ix[s])
            pltpu.async_copy(w_hbm.at[pl.ds(base, K)], w_v[s], sem_ix[s])

        def wait_idx(s):
            pltpu.make_async_copy(ii_hbm.at[pl.ds(0, K)], ii_v[s], sem_ix[s]).wait()
            pltpu.make_async_copy(jj_hbm.at[pl.ds(0, K)], jj_v[s], sem_ix[s]).wait()
            pltpu.make_async_copy(w_hbm.at[pl.ds(0, K)], w_v[s], sem_ix[s]).wait()

        def prep(s, s4):
            @pl.loop(0, K // L)
            def _(t):
                sl = pl.ds(t * L, L)
                wv = w_v[s][sl]
                w2_v[s][sl] = wv * wv
                iv = ii_v[s][sl]
                jv = jj_v[s][sl]
                gij_v[s][sl] = iv + off
                gij_v[s][pl.ds(K + t * L, L)] = jv + off
                si_v[s4][sl] = iv
                sj_v[s4][sl] = jv

        def fire_gather(s):
            pltpu.async_copy(xt_hbm.at[gij_v[s]], gAB[s], sem_g[s])

        def wait_gather(s):
            pltpu.make_async_copy(xt_hbm.at[gij_v[s]], gAB[s], sem_g[s]).wait()

        def fire_scatter(s, s4):
            pltpu.async_copy(dA, acc_sh.at[si_v[s4]], sem_sc[s], add=True)
            pltpu.async_copy(dB, acc_sh.at[sj_v[s4]], sem_sc[s], add=True)

        def wait_scatter(s, s4):
            pltpu.make_async_copy(dA, acc_sh.at[si_v[s4]], sem_sc[s]).wait()
            pltpu.make_async_copy(dB, acc_sh.at[sj_v[s4]], sem_sc[s]).wait()

        def compute(s):
            w2r = w2_v[s]
            gr = gAB[s]
            NCH = 128 // L

            @pl.loop(0, K, step=2)
            def _(e0):
                e1 = e0 + 1
                w2b0 = plsc.load_gather(w2r, [jnp.zeros((L,), jnp.int32) + e0])
                w2b1 = plsc.load_gather(w2r, [jnp.zeros((L,), jnp.int32) + e1])

                def emit(e, w2b, c, a, b):
                    sl = pl.ds(c * L, L)
                    d = w2b * (a - b)
                    dA[e, sl] = d
                    dB[e, sl] = -d

                prev = None
                for c in range(NCH):
                    sl = pl.ds(c * L, L)
                    a0 = gr[e0, sl]
                    b0 = gr[K + e0, sl]
                    a1 = gr[e1, sl]
                    b1 = gr[K + e1, sl]
                    if prev is not None:
                        pc, pa0, pb0, pa1, pb1 = prev
                        emit(e0, w2b0, pc, pa0, pb0)
                        emit(e1, w2b1, pc, pa1, pb1)
                    prev = (c, a0, b0, a1, b1)
                pc, pa0, pb0, pa1, pb1 = prev
                emit(e0, w2b0, pc, pa0, pb0)
                emit(e1, w2b1, pc, pa1, pb1)

        @pl.loop(0, RZ)
        def _(r):
            for c in range(128 // L):
                dA[r, pl.ds(c * L, L)] = zeros

        rows_per_tile = NP // NS
        @pl.loop(0, rows_per_tile // RZ)
        def _(k):
            pltpu.sync_copy(
                dA.at[pl.ds(0, RZ)],
                acc_sh.at[pl.ds(sid * rows_per_tile + k * RZ, RZ)])

        load_idx_sync(0, 0)
        load_idx_sync(1, 1)
        prep(0, 0)
        fire_gather(0)
        plsc.subcore_barrier()

        @pl.loop(0, n_chunks // 4)
        def _(k4):
            for p in range(4):
                k = k4 * 4 + p
                p2 = p % 2
                q2 = 1 - p2

                wait_gather(p2)

                @pl.when(jnp.logical_and(k >= 1, k + 1 < n_chunks))
                def _():
                    wait_idx(q2)

                @pl.when(k + 1 < n_chunks)
                def _():
                    prep(q2, (p + 1) % 4)
                    fire_gather(q2)

                @pl.when(k + 2 < n_chunks)
                def _():
                    load_idx_async(k + 2, p2)

                @pl.when(k >= 1)
                def _():
                    wait_scatter(q2, (p + 3) % 4)

                compute(p2)
                fire_scatter(p2, p)

        wait_scatter((n_chunks - 1) % 2, (n_chunks - 1) % 4)
        plsc.subcore_barrier()

        @pl.loop(0, rows_per_tile // RZ)
        def _(k):
            r0 = sid * rows_per_tile + k * RZ
            pltpu.sync_copy(acc_sh.at[pl.ds(r0, RZ)], dA.at[pl.ds(0, RZ)])
            pltpu.sync_copy(dA.at[pl.ds(0, RZ)],
                            out_hbm.at[pl.ds(cid * NP + r0, RZ)])

    return graph_lap


def kernel(x, W, iInd, jInd):
    B, C, N = x.shape
    E = iInd.shape[0]
    CH = C // NC

    n_chunks = -(-E // (NS * K))
    n_chunks = -(-n_chunks // 4) * 4
    E_pad = NS * K * n_chunks
    pad = E_pad - E
    ii = jnp.concatenate([iInd.astype(jnp.int32), jnp.zeros((pad,), jnp.int32)])
    jj = jnp.concatenate([jInd.astype(jnp.int32), jnp.zeros((pad,), jnp.int32)])
    w = jnp.concatenate([W[0, 0].astype(jnp.float32), jnp.zeros((pad,), jnp.float32)])

    NP = -(-N // (NS * RZ)) * (NS * RZ)
    xt = x[0].T.reshape(N, NC, CH).transpose(1, 0, 2).reshape(NC * N, CH)
    xt = jnp.pad(xt.reshape(NC, N, CH), ((0, 0), (0, NP - N), (0, 0))).reshape(NC * NP, CH)

    out2 = _build(NP, CH, n_chunks)(xt, w, ii, jj)
    return out2.reshape(NC, NP, CH)[:, :N].transpose(0, 2, 1).reshape(1, C, N)
